# Initial kernel scaffold; baseline (speedup 1.0000x reference)
import jax
import jax.numpy as jnp
from jax.experimental import pallas as pl


def kernel(x, emb, W, b):
    raise NotImplementedError("write your pallas kernel here")


# speedup vs baseline: 8.4950x; 8.4950x over previous
import functools

import jax
import jax.numpy as jnp
from jax import lax
from jax.experimental import pallas as pl
from jax.experimental.pallas import tpu as pltpu
from jax.experimental.pallas import tpu_sc as plsc

_VOCAB = 100000
_D = 64
_L = 50
_B = 4096

_NC = 2
_NS = 16
_NW = _NC * _NS
_BPW = _B // _NW
_CB = 16
_NCHUNK = _BPW // _CB
_LANES = 16
_DG = _D // _LANES


def _sc_body(x_hbm, emb_hbm, w_hbm, out_hbm, idx_v, rows_v, w_v, out_v, sem):
    wid = lax.axis_index("s") * _NC + lax.axis_index("c")

    pltpu.sync_copy(w_hbm, w_v)

    iota = lax.iota(jnp.int32, _LANES)

    for ci in range(_NCHUNK):
        base = wid * _BPW * _L + ci * _CB * _L
        pltpu.sync_copy(x_hbm.at[pl.ds(base, _CB * _L)], idx_v)
        pltpu.async_copy(emb_hbm.at[idx_v], rows_v, sem).wait()

        def l_body(l, accs):
            accs = list(accs)
            for d in range(_DG):
                wv = w_v[l, d * _LANES:(d + 1) * _LANES]
                for r in range(_CB):
                    accs[r] = accs[r] + rows_v[r * _L + l,
                                               d * _LANES:(d + 1) * _LANES] * wv
            return tuple(accs)

        zero = jnp.zeros((_LANES,), jnp.float32)
        accs = lax.fori_loop(0, _L, l_body, (zero,) * _CB)

        def lane_swap(v, s):
            return v.at[iota ^ s].get(mode="promise_in_bounds")

        def merge(a, b, s):
            return jnp.where((iota & s) == 0,
                             a + lane_swap(a, s), b + lane_swap(b, s))

        vs = list(accs)
        for s in (1, 2, 4, 8):
            vs = [merge(vs[2 * i], vs[2 * i + 1], s)
                  for i in range(len(vs) // 2)]
        out_v[pl.ds(ci * _CB, _CB)] = vs[0]

    pltpu.sync_copy(out_v, out_hbm.at[pl.ds(wid * _BPW, _BPW)])


@functools.partial(jax.jit, static_argnames=())
def _run(x_flat, emb, w2d):
    mesh = plsc.VectorSubcoreMesh(core_axis_name="c", subcore_axis_name="s")
    f = pl.kernel(
        _sc_body,
        out_type=jax.ShapeDtypeStruct((_B,), jnp.float32),
        mesh=mesh,
        scratch_types=[
            pltpu.VMEM((_CB * _L,), jnp.int32),
            pltpu.VMEM((_CB * _L, _D), jnp.float32),
            pltpu.VMEM((_L, _D), jnp.float32),
            pltpu.VMEM((_BPW,), jnp.float32),
            pltpu.SemaphoreType.DMA,
        ],
        compiler_params=pltpu.CompilerParams(use_tc_tiling_on_sc=False),
    )
    return f(x_flat, emb, w2d)


def kernel(x, emb, W, b):
    x_flat = x.reshape(-1).astype(jnp.int32)
    w2d = W.reshape(_L, _D)
    out = _run(x_flat, emb, w2d)
    return out + b[0]
